# baseline (speedup 1.0000x reference)
import functools

import jax
import jax.numpy as jnp
from jax import lax
from jax.experimental import pallas as pl
from jax.experimental.pallas import tpu as pltpu
from jax.experimental.pallas import tpu_sc as plsc

_NC = 2
_NS = 16
_NW = _NC * _NS
_L = 16
_CH = 256


@functools.partial(jax.jit, static_argnums=(2, 3, 4, 5))
def _emb_call(idx, tbl_flat, B, V, D, b_per_w):
  n_chunks = b_per_w // _CH
  mesh = plsc.VectorSubcoreMesh(core_axis_name="c", subcore_axis_name="s")

  @functools.partial(
      pl.kernel,
      mesh=mesh,
      out_type=jax.ShapeDtypeStruct((B, D), jnp.float32),
      compiler_params=pltpu.CompilerParams(needs_layout_passes=False),
      scratch_types=[
          pltpu.VMEM((b_per_w,), jnp.int32),
          pltpu.VMEM((_CH, D), jnp.float32),
          pltpu.VMEM((_CH, D), jnp.float32),
          pltpu.VMEM((V * D,), jnp.float32),
          pltpu.SemaphoreType.DMA,
          pltpu.SemaphoreType.DMA,
      ],
  )
  def emb(idx_hbm, tbl_hbm, out_hbm, idx_v, rows0, rows1, tbl_v, sem0, sem1):
    wid = lax.axis_index("s") * _NC + lax.axis_index("c")
    base = wid * b_per_w
    pltpu.sync_copy(tbl_hbm, tbl_v)
    pltpu.sync_copy(idx_hbm.at[pl.ds(base, b_per_w)], idx_v)

    gdn = lax.GatherDimensionNumbers(
        offset_dims=(), collapsed_slice_dims=(0,), start_index_map=(0,))

    def lane_bcast(xv, i):
      idx = jnp.full((_L, 1), i, jnp.int32)
      return lax.gather(xv, idx, gdn, slice_sizes=(1,),
                        mode=lax.GatherScatterMode.PROMISE_IN_BOUNDS)

    iota = lax.iota(jnp.int32, _L)
    bufs = (rows0, rows1)
    sems = (sem0, sem1)

    def compute_chunk(c, rows_v):
      nb = D // _L
      vals = None
      for g in range(_CH // _L):
        xv = idx_v[pl.ds(c * _CH + g * _L, _L)]
        for i in range(_L):
          row = g * _L + i
          gb = lane_bcast(xv, i) * D + iota
          nxt = []
          for b in range(nb):
            if vals is not None:
              rows_v[row - 1, pl.ds(b * _L, _L)] = vals[b]
            nxt.append(plsc.load_gather(tbl_v, [gb + (b * _L)]))
          vals = nxt
      for b in range(nb):
        rows_v[_CH - 1, pl.ds(b * _L, _L)] = vals[b]

    def pair_body(c2, carry):
      for b in range(2):
        c = c2 * 2 + b

        @pl.when(c2 > 0)
        def _wait():
          pltpu.make_async_copy(bufs[b], out_hbm.at[pl.ds(0, _CH)], sems[b]).wait()

        compute_chunk(c, bufs[b])
        pltpu.async_copy(bufs[b], out_hbm.at[pl.ds(base + c * _CH, _CH)], sems[b])
      return carry

    lax.fori_loop(0, n_chunks // 2, pair_body, 0)
    for b in range(2):
      pltpu.make_async_copy(bufs[b], out_hbm.at[pl.ds(0, _CH)], sems[b]).wait()

  return emb(idx, tbl_flat)


def kernel(X, table):
  B = X.size
  V, D = table.shape
  idx = X.reshape(B).astype(jnp.int32)
  b_per_w = B // _NW
  out = _emb_call(idx, table.reshape(V * D), B, V, D, b_per_w)
  return out.reshape(X.shape + (D,))


# speedup vs baseline: 2.3998x; 2.3998x over previous
import functools

import jax
import jax.numpy as jnp
from jax import lax
from jax.experimental import pallas as pl
from jax.experimental.pallas import tpu as pltpu
from jax.experimental.pallas import tpu_sc as plsc

_NC = 2
_NS = 16
_NW = _NC * _NS
_L = 16
_CH = 256
_G = 32


@functools.partial(jax.jit, static_argnums=(2, 3, 4, 5))
def _emb_call(idx, tbl_flat, B, V, D, b_per_w):
  n_chunks = b_per_w // _CH
  mesh = plsc.VectorSubcoreMesh(core_axis_name="c", subcore_axis_name="s")

  @functools.partial(
      pl.kernel,
      mesh=mesh,
      out_type=jax.ShapeDtypeStruct((B, D), jnp.float32),
      compiler_params=pltpu.CompilerParams(needs_layout_passes=False),
      scratch_types=[
          pltpu.VMEM((b_per_w,), jnp.int32),
          pltpu.VMEM((_CH, D), jnp.float32),
          pltpu.VMEM((_CH, D), jnp.float32),
          pltpu.VMEM((V * D,), jnp.float32),
          pltpu.SemaphoreType.DMA,
          pltpu.SemaphoreType.DMA,
      ],
  )
  def emb(idx_hbm, tbl_hbm, out_hbm, idx_v, rows0, rows1, tbl_v, sem0, sem1):
    wid = lax.axis_index("s") * _NC + lax.axis_index("c")
    base = wid * b_per_w
    pltpu.sync_copy(tbl_hbm, tbl_v)
    pltpu.sync_copy(idx_hbm.at[pl.ds(base, b_per_w)], idx_v)

    gdn = lax.GatherDimensionNumbers(
        offset_dims=(), collapsed_slice_dims=(0,), start_index_map=(0,))

    def lane_bcast(xv, i):
      idx = jnp.full((_L, 1), i, jnp.int32)
      return lax.gather(xv, idx, gdn, slice_sizes=(1,),
                        mode=lax.GatherScatterMode.PROMISE_IN_BOUNDS)

    iota = lax.iota(jnp.int32, _L)
    bufs = (rows0, rows1)
    sems = (sem0, sem1)

    def compute_chunk(c, rows_v):
      nb = D // _L
      ng = _CH // _G

      def group_body(g, carry):
        base = g * _G
        vals = None
        xv = None
        for i in range(_G):
          if i % _L == 0:
            xv = idx_v[pl.ds(c * _CH + base + i, _L)]
          row = base + i
          gb = lane_bcast(xv, i % _L) * D + iota
          nxt = []
          for b in range(nb):
            if vals is not None:
              rows_v[row - 1, pl.ds(b * _L, _L)] = vals[b]
            nxt.append(plsc.load_gather(tbl_v, [gb + (b * _L)]))
          vals = nxt
        for b in range(nb):
          rows_v[base + _G - 1, pl.ds(b * _L, _L)] = vals[b]
        return carry

      lax.fori_loop(0, ng, group_body, 0)

    def pair_body(c2, carry):
      for b in range(2):
        c = c2 * 2 + b

        @pl.when(c2 > 0)
        def _wait():
          pltpu.make_async_copy(bufs[b], out_hbm.at[pl.ds(0, _CH)], sems[b]).wait()

        compute_chunk(c, bufs[b])
        pltpu.async_copy(bufs[b], out_hbm.at[pl.ds(base + c * _CH, _CH)], sems[b])
      return carry

    lax.fori_loop(0, n_chunks // 2, pair_body, 0)
    for b in range(2):
      pltpu.make_async_copy(bufs[b], out_hbm.at[pl.ds(0, _CH)], sems[b]).wait()

  return emb(idx, tbl_flat)


def kernel(X, table):
  B = X.size
  V, D = table.shape
  idx = X.reshape(B).astype(jnp.int32)
  b_per_w = B // _NW
  out = _emb_call(idx, table.reshape(V * D), B, V, D, b_per_w)
  return out.reshape(X.shape + (D,))
